# baseline (speedup 1.0000x reference)
import functools

import jax
import jax.numpy as jnp
from jax import lax
from jax.experimental import pallas as pl
from jax.experimental.pallas import tpu as pltpu
from jax.experimental.pallas import tpu_sc as plsc

_VOCAB = 1000
_HIDDEN = 128
_BATCH = 1024
_SEQ = 50

_NC, _NS = 2, 16
_NW = _NC * _NS
_BPW = _BATCH // _NW
_NBUF = 2


def _table_body(e_ref, wt_ref, b_ref, m_ref):
    m_ref[...] = (
        jnp.dot(e_ref[...], wt_ref[...], preferred_element_type=jnp.float32)
        + b_ref[...]
    )


def _gather_body(table_hbm, idx_hbm, out_hbm, idx_v, rows_v, g0, g1, w0, w1):
    gsems = (g0, g1)
    wsems = (w0, w1)
    wid = lax.axis_index("s") * _NC + lax.axis_index("c")
    b0 = wid * _BPW
    pltpu.sync_copy(idx_hbm.at[pl.ds(b0, _BPW)], idx_v)

    def start_gather(jb, b):
        pltpu.async_copy(
            table_hbm.at[idx_v.at[jb]], rows_v.at[b], gsems[b])

    def wait_gather(b):
        pltpu.make_async_copy(
            table_hbm.at[pl.ds(0, _SEQ)], rows_v.at[b], gsems[b]).wait()

    def start_write(jb, b):
        pltpu.async_copy(rows_v.at[b], out_hbm.at[b0 + jb], wsems[b])

    def wait_write(b):
        pltpu.make_async_copy(
            rows_v.at[b], out_hbm.at[0], wsems[b]).wait()

    start_gather(0, 0)
    for jb in range(_BPW):
        b = jb % _NBUF
        if jb + 1 < _BPW:
            if jb >= 1:
                wait_write(1 - b)
            start_gather(jb + 1, 1 - b)
        wait_gather(b)
        start_write(jb, b)
    for b in range(_NBUF):
        wait_write(b)


def kernel(input_ids, embed_table, head_w, head_b):
    table = pl.pallas_call(
        _table_body,
        out_shape=jax.ShapeDtypeStruct((_VOCAB, _VOCAB), jnp.float32),
    )(embed_table, head_w.T, head_b.reshape(1, _VOCAB))

    idx = input_ids.astype(jnp.int32)
    mesh = plsc.VectorSubcoreMesh(
        core_axis_name="c", subcore_axis_name="s",
        num_cores=_NC, num_subcores=_NS,
    )
    out = pl.kernel(
        _gather_body,
        out_type=jax.ShapeDtypeStruct((_BATCH, _SEQ, _VOCAB), jnp.float32),
        mesh=mesh,
        compiler_params=pltpu.CompilerParams(use_tc_tiling_on_sc=False),
        scratch_types=[
            pltpu.VMEM((_BPW, _SEQ), jnp.int32),
            pltpu.VMEM((_NBUF, _SEQ, _VOCAB), jnp.float32),
        ] + [pltpu.SemaphoreType.DMA] * (2 * _NBUF),
    )(table, idx)
    return out
